# Initial kernel scaffold; baseline (speedup 1.0000x reference)
import jax
import jax.numpy as jnp
from jax.experimental import pallas as pl


def kernel(atomic_numbers, neighbour_indices, neighbour_displacements, embed_table, W_emb, b_emb, rad_mix, Wtd0a, Wtd0b, Wtd1a, Wtd1b):
    raise NotImplementedError("write your pallas kernel here")


# speedup vs baseline: 3.1109x; 3.1109x over previous
import functools

import jax
import jax.numpy as jnp
import numpy as np
from jax.experimental import pallas as pl
from jax.experimental.pallas import tpu as pltpu

N_ATOMS = 10000
N_EDGES = 160000
N_SPECIES = 100
F = 64
N_RBF = 32
N_SH = 9
CUTOFF = 5.0

BE = 2000
GRID = N_EDGES // BE


def _sh_channels(x, y, z):
    return [
        jnp.full_like(x, 0.28209479),
        0.48860251 * y,
        0.48860251 * z,
        0.48860251 * x,
        1.09254843 * x * y,
        1.09254843 * y * z,
        0.31539157 * (3.0 * z * z - 1.0),
        1.09254843 * x * z,
        0.54627422 * (x * x - y * y),
    ]


def _edge_kernel(idx_i_ref, disp_ref, zi_ref, zj_ref, zatoms_ref,
                 mix_ref, table2_ref, w0a_ref, w0b_ref, w1a_ref, w1b_ref,
                 out_ref, y_scr):
    g = pl.program_id(0)

    @pl.when(g == 0)
    def _init():
        out_ref[...] = jnp.zeros_like(out_ref)

    disp = disp_ref[...]
    dx = disp[:, 0:1]
    dy = disp[:, 1:2]
    dz = disp[:, 2:3]
    r2 = dx * dx + dy * dy + dz * dz
    r = jnp.sqrt(r2)
    inv = 1.0 / (r + 1e-9)
    ux, uy, uz = dx * inv, dy * inv, dz * inv

    centers = jax.lax.broadcasted_iota(jnp.int32, (1, N_RBF), 1).astype(
        jnp.float32) * (CUTOFF / (N_RBF - 1))
    d = r - centers
    env = 0.5 * (jnp.cos(jnp.pi * jnp.clip(r / CUTOFF, 0.0, 1.0)) + 1.0)
    rbf = jnp.exp(-4.0 * d * d) * env

    zj = zj_ref[...]
    def mix_body(s, acc):
        m = mix_ref[s]
        contrib = jnp.dot(rbf, m, preferred_element_type=jnp.float32)
        mask = (zj == s).astype(jnp.float32)
        return acc + mask * contrib
    radial = jax.lax.fori_loop(
        0, N_SPECIES, mix_body, jnp.zeros((BE, F), jnp.float32))

    zi = zi_ref[...]
    sp_iota = jax.lax.broadcasted_iota(jnp.int32, (1, N_SPECIES), 1)
    onehot_i = (zi == sp_iota).astype(jnp.float32)
    emb_i = jnp.dot(onehot_i, table2_ref[...],
                    preferred_element_type=jnp.float32)

    sh = _sh_channels(ux, uy, uz)
    ys = [sh[c] * radial for c in range(9)]

    for wa_ref, wb_ref in ((w0a_ref, w0b_ref), (w1a_ref, w1b_ref)):
        wa = wa_ref[...]
        wb = wb_ref[...]
        hs = [jnp.dot(yc, wa, preferred_element_type=jnp.float32)
              for yc in ys]
        gate = hs[0] * jax.nn.sigmoid(hs[0])
        ys = [jnp.dot(hc * gate, wb, preferred_element_type=jnp.float32)
              for hc in hs]

    for c in range(9):
        y_scr[:, c * F:(c + 1) * F] = ys[c] * emb_i

    def scat_body(e, _):
        idx = idx_i_ref[0, 0, e]
        out_ref[pl.ds(idx, 1), :] += y_scr[pl.ds(e, 1), :]
        return 0
    jax.lax.fori_loop(0, BE, scat_body, 0)

    @pl.when(g == GRID - 1)
    def _residual():
        za = zatoms_ref[...]
        onehot_a = (za == sp_iota).astype(jnp.float32)
        res = jnp.dot(onehot_a, table2_ref[...],
                      preferred_element_type=jnp.float32)
        out_ref[:, 0:F] += res


def kernel(atomic_numbers, neighbour_indices, neighbour_displacements,
           embed_table, W_emb, b_emb, rad_mix, Wtd0a, Wtd0b, Wtd1a, Wtd1b):
    idx_i = neighbour_indices[:, 0]
    idx_j = neighbour_indices[:, 1]
    zi = jnp.take(atomic_numbers, idx_i, axis=0).astype(jnp.int32)
    zj = jnp.take(atomic_numbers, idx_j, axis=0).astype(jnp.int32)
    table2 = embed_table @ W_emb + b_emb

    out = pl.pallas_call(
        _edge_kernel,
        grid=(GRID,),
        in_specs=[
            pl.BlockSpec((1, 1, BE), lambda i: (i, 0, 0),
                         memory_space=pltpu.SMEM),
            pl.BlockSpec((BE, 3), lambda i: (i, 0)),
            pl.BlockSpec((BE, 1), lambda i: (i, 0)),
            pl.BlockSpec((BE, 1), lambda i: (i, 0)),
            pl.BlockSpec((N_ATOMS, 1), lambda i: (0, 0)),
            pl.BlockSpec((N_SPECIES, N_RBF, F), lambda i: (0, 0, 0)),
            pl.BlockSpec((N_SPECIES, F), lambda i: (0, 0)),
            pl.BlockSpec((F, F), lambda i: (0, 0)),
            pl.BlockSpec((F, F), lambda i: (0, 0)),
            pl.BlockSpec((F, F), lambda i: (0, 0)),
            pl.BlockSpec((F, F), lambda i: (0, 0)),
        ],
        out_specs=pl.BlockSpec((N_ATOMS, N_SH * F), lambda i: (0, 0)),
        out_shape=jax.ShapeDtypeStruct((N_ATOMS, N_SH * F), jnp.float32),
        scratch_shapes=[pltpu.VMEM((BE, N_SH * F), jnp.float32)],
    )(idx_i.astype(jnp.int32).reshape(GRID, 1, BE),
      neighbour_displacements,
      zi.reshape(N_EDGES, 1),
      zj.reshape(N_EDGES, 1),
      atomic_numbers.astype(jnp.int32).reshape(N_ATOMS, 1),
      rad_mix, table2, Wtd0a, Wtd0b, Wtd1a, Wtd1b)

    return out.reshape(N_ATOMS, N_SH, F)


# speedup vs baseline: 3.5179x; 1.1308x over previous
import functools

import jax
import jax.numpy as jnp
import numpy as np
from jax import lax
from jax.experimental import pallas as pl
from jax.experimental.pallas import tpu as pltpu
from jax.experimental.pallas import tpu_sc as plsc

N_ATOMS = 10000
N_EDGES = 160000
N_SPECIES = 100
F = 64
N_RBF = 32
N_SH = 9
D_OUT = N_SH * F
CUTOFF = 5.0

BE = 2000
GRID = N_EDGES // BE

NCORES = 2
NSUB = 16
NCH = 5
CCH = 128
N_PAD = 10240
ROWS_PT = N_PAD // NSUB
EPT = N_EDGES // NSUB
BATCH = 128
NSTEP = EPT // BATCH
TAIL = EPT - NSTEP * BATCH


def _sh_channels(x, y, z):
    return [
        jnp.full_like(x, 0.28209479),
        0.48860251 * y,
        0.48860251 * z,
        0.48860251 * x,
        1.09254843 * x * y,
        1.09254843 * y * z,
        0.31539157 * (3.0 * z * z - 1.0),
        1.09254843 * x * z,
        0.54627422 * (x * x - y * y),
    ]


def _edge_kernel(disp_ref, zi_ref, zj_ref, zatoms_ref,
                 mix_ref, table2_ref, w0a_ref, w0b_ref, w1a_ref, w1b_ref,
                 y_ref, res_ref):
    g = pl.program_id(0)

    disp = disp_ref[...]
    dx = disp[:, 0:1]
    dy = disp[:, 1:2]
    dz = disp[:, 2:3]
    r2 = dx * dx + dy * dy + dz * dz
    r = jnp.sqrt(r2)
    inv = 1.0 / (r + 1e-9)
    ux, uy, uz = dx * inv, dy * inv, dz * inv

    centers = jax.lax.broadcasted_iota(jnp.int32, (1, N_RBF), 1).astype(
        jnp.float32) * (CUTOFF / (N_RBF - 1))
    d = r - centers
    env = 0.5 * (jnp.cos(jnp.pi * jnp.clip(r / CUTOFF, 0.0, 1.0)) + 1.0)
    rbf = jnp.exp(-4.0 * d * d) * env

    zj = zj_ref[...]
    def mix_body(s, acc):
        m = mix_ref[s]
        contrib = jnp.dot(rbf, m, preferred_element_type=jnp.float32)
        mask = (zj == s).astype(jnp.float32)
        return acc + mask * contrib
    radial = jax.lax.fori_loop(
        0, N_SPECIES, mix_body, jnp.zeros((BE, F), jnp.float32))

    zi = zi_ref[...]
    sp_iota = jax.lax.broadcasted_iota(jnp.int32, (1, N_SPECIES), 1)
    onehot_i = (zi == sp_iota).astype(jnp.float32)
    emb_i = jnp.dot(onehot_i, table2_ref[...],
                    preferred_element_type=jnp.float32)

    sh = _sh_channels(ux, uy, uz)
    ys = [sh[c] * radial for c in range(9)]

    for wa_ref, wb_ref in ((w0a_ref, w0b_ref), (w1a_ref, w1b_ref)):
        wa = wa_ref[...]
        wb = wb_ref[...]
        hs = [jnp.dot(yc, wa, preferred_element_type=jnp.float32)
              for yc in ys]
        gate = hs[0] * jax.nn.sigmoid(hs[0])
        ys = [jnp.dot(hc * gate, wb, preferred_element_type=jnp.float32)
              for hc in hs]

    for ch in range(4):
        for j in range(2):
            y_ref[ch, :, j * F:(j + 1) * F] = ys[2 * ch + j] * emb_i
    y_ref[4, :, 0:F] = ys[8] * emb_i
    y_ref[4, :, F:CCH] = jnp.zeros((BE, CCH - F), jnp.float32)

    @pl.when(g == GRID - 1)
    def _residual():
        za = zatoms_ref[...]
        onehot_a = (za == sp_iota).astype(jnp.float32)
        res = jnp.dot(onehot_a, table2_ref[...],
                      preferred_element_type=jnp.float32)
        res_ref[...] = jnp.concatenate(
            [res, jnp.zeros((N_PAD, CCH - F), jnp.float32)], axis=1)


def _sc_scatter(y_hbm, idx_hbm, res_hbm, zero_hbm, out_hbm,
                idx_v, y_v, idx_t, y_t, acc):
    c = lax.axis_index("c")
    s = lax.axis_index("s")
    stripe = pl.ds(s * ROWS_PT, ROWS_PT)

    for k in range(3):
        chunk = c + NCORES * k

        @pl.when(chunk < NCH)
        def _pass():
            @pl.when(chunk == 0)
            def _init_res():
                pltpu.sync_copy(res_hbm.at[stripe], acc.at[stripe])

            @pl.when(chunk != 0)
            def _init_zero():
                pltpu.sync_copy(zero_hbm, acc.at[stripe])

            plsc.subcore_barrier()

            def step(i, _):
                base = s * EPT + i * BATCH
                pltpu.sync_copy(idx_hbm.at[pl.ds(base, BATCH)], idx_v)
                pltpu.sync_copy(y_hbm.at[chunk, pl.ds(base, BATCH)], y_v)
                pltpu.sync_copy(y_v, acc.at[idx_v], add=True)
                return 0
            lax.fori_loop(0, NSTEP, step, 0)

            tbase = s * EPT + NSTEP * BATCH
            pltpu.sync_copy(idx_hbm.at[pl.ds(tbase, TAIL)], idx_t)
            pltpu.sync_copy(y_hbm.at[chunk, pl.ds(tbase, TAIL)], y_t)
            pltpu.sync_copy(y_t, acc.at[idx_t], add=True)

            plsc.subcore_barrier()

            pltpu.sync_copy(acc.at[stripe], out_hbm.at[chunk, stripe])


def kernel(atomic_numbers, neighbour_indices, neighbour_displacements,
           embed_table, W_emb, b_emb, rad_mix, Wtd0a, Wtd0b, Wtd1a, Wtd1b):
    idx_i = neighbour_indices[:, 0].astype(jnp.int32)
    idx_j = neighbour_indices[:, 1]
    zi = jnp.take(atomic_numbers, idx_i, axis=0).astype(jnp.int32)
    zj = jnp.take(atomic_numbers, idx_j, axis=0).astype(jnp.int32)
    table2 = embed_table @ W_emb + b_emb
    zatoms = jnp.concatenate(
        [atomic_numbers.astype(jnp.int32),
         jnp.full((N_PAD - N_ATOMS,), -1, jnp.int32)])

    y, res = pl.pallas_call(
        _edge_kernel,
        grid=(GRID,),
        in_specs=[
            pl.BlockSpec((BE, 3), lambda i: (i, 0)),
            pl.BlockSpec((BE, 1), lambda i: (i, 0)),
            pl.BlockSpec((BE, 1), lambda i: (i, 0)),
            pl.BlockSpec((N_PAD, 1), lambda i: (0, 0)),
            pl.BlockSpec((N_SPECIES, N_RBF, F), lambda i: (0, 0, 0)),
            pl.BlockSpec((N_SPECIES, F), lambda i: (0, 0)),
            pl.BlockSpec((F, F), lambda i: (0, 0)),
            pl.BlockSpec((F, F), lambda i: (0, 0)),
            pl.BlockSpec((F, F), lambda i: (0, 0)),
            pl.BlockSpec((F, F), lambda i: (0, 0)),
        ],
        out_specs=[
            pl.BlockSpec((NCH, BE, CCH), lambda i: (0, i, 0)),
            pl.BlockSpec((N_PAD, CCH), lambda i: (0, 0)),
        ],
        out_shape=[
            jax.ShapeDtypeStruct((NCH, N_EDGES, CCH), jnp.float32),
            jax.ShapeDtypeStruct((N_PAD, CCH), jnp.float32),
        ],
    )(neighbour_displacements,
      zi.reshape(N_EDGES, 1),
      zj.reshape(N_EDGES, 1),
      zatoms.reshape(N_PAD, 1),
      rad_mix, table2, Wtd0a, Wtd0b, Wtd1a, Wtd1b)

    zero = jnp.zeros((ROWS_PT, CCH), jnp.float32)
    mesh = plsc.VectorSubcoreMesh(core_axis_name="c", subcore_axis_name="s")
    out = pl.kernel(
        _sc_scatter,
        mesh=mesh,
        compiler_params=pltpu.CompilerParams(use_tc_tiling_on_sc=False),
        out_type=jax.ShapeDtypeStruct((NCH, N_PAD, CCH), jnp.float32),
        scratch_types=[
            pltpu.VMEM((BATCH,), jnp.int32),
            pltpu.VMEM((BATCH, CCH), jnp.float32),
            pltpu.VMEM((TAIL,), jnp.int32),
            pltpu.VMEM((TAIL, CCH), jnp.float32),
            pltpu.VMEM_SHARED((N_PAD, CCH), jnp.float32),
        ],
    )(y, idx_i, res, zero)

    flat = out.transpose(1, 0, 2).reshape(N_PAD, NCH * CCH)
    return flat[:N_ATOMS, :D_OUT].reshape(N_ATOMS, N_SH, F)


# speedup vs baseline: 4.7683x; 1.3555x over previous
import functools

import jax
import jax.numpy as jnp
import numpy as np
from jax import lax
from jax.experimental import pallas as pl
from jax.experimental.pallas import tpu as pltpu
from jax.experimental.pallas import tpu_sc as plsc

N_ATOMS = 10000
N_EDGES = 160000
N_SPECIES = 100
F = 64
N_RBF = 32
N_SH = 9
D_OUT = N_SH * F
CUTOFF = 5.0

BE = 1000
GRID = N_EDGES // BE

NCORES = 2
NSUB = 16
NCH = 5
CCH = 128
N_PAD = 10240
ROWS_PT = N_PAD // NSUB
EPT = N_EDGES // NSUB
BATCH = 128
NSTEP = EPT // BATCH
TAIL = EPT - NSTEP * BATCH


def _sh_channels(x, y, z):
    return [
        jnp.full_like(x, 0.28209479),
        0.48860251 * y,
        0.48860251 * z,
        0.48860251 * x,
        1.09254843 * x * y,
        1.09254843 * y * z,
        0.31539157 * (3.0 * z * z - 1.0),
        1.09254843 * x * z,
        0.54627422 * (x * x - y * y),
    ]


def _edge_kernel(disp_ref, zi_ref, zj_ref, zatoms_ref,
                 mix_ref, table2_ref, w0a_ref, w0b_ref, w1a_ref, w1b_ref,
                 y_ref, res_ref):
    g = pl.program_id(0)

    disp = disp_ref[...]
    dx = disp[:, 0:1]
    dy = disp[:, 1:2]
    dz = disp[:, 2:3]
    r2 = dx * dx + dy * dy + dz * dz
    r = jnp.sqrt(r2)
    inv = 1.0 / (r + 1e-9)
    ux, uy, uz = dx * inv, dy * inv, dz * inv

    centers = jax.lax.broadcasted_iota(jnp.int32, (1, N_RBF), 1).astype(
        jnp.float32) * (CUTOFF / (N_RBF - 1))
    d = r - centers
    env = 0.5 * (jnp.cos(jnp.pi * jnp.clip(r / CUTOFF, 0.0, 1.0)) + 1.0)
    rbf = jnp.exp(-4.0 * d * d) * env

    zj = zj_ref[...]
    sp_iota = jax.lax.broadcasted_iota(jnp.int32, (1, N_SPECIES), 1)
    onehot_j = (zj == sp_iota).astype(jnp.float32)
    radial_all = jnp.dot(onehot_j, mix_ref[...],
                         preferred_element_type=jnp.float32)
    radial = rbf[:, 0:1] * radial_all[:, 0:F]
    for b in range(1, N_RBF):
        radial = radial + rbf[:, b:b + 1] * radial_all[:, b * F:(b + 1) * F]

    zi = zi_ref[...]
    onehot_i = (zi == sp_iota).astype(jnp.float32)
    emb_i = jnp.dot(onehot_i, table2_ref[...],
                    preferred_element_type=jnp.float32)

    sh = _sh_channels(ux, uy, uz)
    ys = [sh[c] * radial for c in range(9)]

    for wa_ref, wb_ref in ((w0a_ref, w0b_ref), (w1a_ref, w1b_ref)):
        wa = wa_ref[...]
        wb = wb_ref[...]
        hs = [jnp.dot(yc, wa, preferred_element_type=jnp.float32)
              for yc in ys]
        gate = hs[0] * jax.nn.sigmoid(hs[0])
        ys = [jnp.dot(hc * gate, wb, preferred_element_type=jnp.float32)
              for hc in hs]

    for ch in range(4):
        for j in range(2):
            y_ref[ch, :, j * F:(j + 1) * F] = ys[2 * ch + j] * emb_i
    y_ref[4, :, 0:F] = ys[8] * emb_i
    y_ref[4, :, F:CCH] = jnp.zeros((BE, CCH - F), jnp.float32)

    @pl.when(g == GRID - 1)
    def _residual():
        za = zatoms_ref[...]
        onehot_a = (za == sp_iota).astype(jnp.float32)
        res = jnp.dot(onehot_a, table2_ref[...],
                      preferred_element_type=jnp.float32)
        res_ref[...] = jnp.concatenate(
            [res, jnp.zeros((N_PAD, CCH - F), jnp.float32)], axis=1)


def _sc_scatter(y_hbm, idx_hbm, res_hbm, zero_hbm, out_hbm,
                idx_v, y_v, idx_t, y_t, acc):
    c = lax.axis_index("c")
    s = lax.axis_index("s")
    stripe = pl.ds(s * ROWS_PT, ROWS_PT)

    for k in range(3):
        chunk = c + NCORES * k

        @pl.when(chunk < NCH)
        def _pass():
            @pl.when(chunk == 0)
            def _init_res():
                pltpu.sync_copy(res_hbm.at[stripe], acc.at[stripe])

            @pl.when(chunk != 0)
            def _init_zero():
                pltpu.sync_copy(zero_hbm, acc.at[stripe])

            plsc.subcore_barrier()

            def step(i, _):
                base = s * EPT + i * BATCH
                pltpu.sync_copy(idx_hbm.at[pl.ds(base, BATCH)], idx_v)
                pltpu.sync_copy(y_hbm.at[chunk, pl.ds(base, BATCH)], y_v)
                pltpu.sync_copy(y_v, acc.at[idx_v], add=True)
                return 0
            lax.fori_loop(0, NSTEP, step, 0)

            tbase = s * EPT + NSTEP * BATCH
            pltpu.sync_copy(idx_hbm.at[pl.ds(tbase, TAIL)], idx_t)
            pltpu.sync_copy(y_hbm.at[chunk, pl.ds(tbase, TAIL)], y_t)
            pltpu.sync_copy(y_t, acc.at[idx_t], add=True)

            plsc.subcore_barrier()

            pltpu.sync_copy(acc.at[stripe], out_hbm.at[chunk, stripe])


def kernel(atomic_numbers, neighbour_indices, neighbour_displacements,
           embed_table, W_emb, b_emb, rad_mix, Wtd0a, Wtd0b, Wtd1a, Wtd1b):
    idx_i = neighbour_indices[:, 0].astype(jnp.int32)
    idx_j = neighbour_indices[:, 1]
    zi = jnp.take(atomic_numbers, idx_i, axis=0).astype(jnp.int32)
    zj = jnp.take(atomic_numbers, idx_j, axis=0).astype(jnp.int32)
    table2 = embed_table @ W_emb + b_emb
    zatoms = jnp.concatenate(
        [atomic_numbers.astype(jnp.int32),
         jnp.full((N_PAD - N_ATOMS,), -1, jnp.int32)])

    y, res = pl.pallas_call(
        _edge_kernel,
        grid=(GRID,),
        in_specs=[
            pl.BlockSpec((BE, 3), lambda i: (i, 0)),
            pl.BlockSpec((BE, 1), lambda i: (i, 0)),
            pl.BlockSpec((BE, 1), lambda i: (i, 0)),
            pl.BlockSpec((N_PAD, 1), lambda i: (0, 0)),
            pl.BlockSpec((N_SPECIES, N_RBF * F), lambda i: (0, 0)),
            pl.BlockSpec((N_SPECIES, F), lambda i: (0, 0)),
            pl.BlockSpec((F, F), lambda i: (0, 0)),
            pl.BlockSpec((F, F), lambda i: (0, 0)),
            pl.BlockSpec((F, F), lambda i: (0, 0)),
            pl.BlockSpec((F, F), lambda i: (0, 0)),
        ],
        out_specs=[
            pl.BlockSpec((NCH, BE, CCH), lambda i: (0, i, 0)),
            pl.BlockSpec((N_PAD, CCH), lambda i: (0, 0)),
        ],
        out_shape=[
            jax.ShapeDtypeStruct((NCH, N_EDGES, CCH), jnp.float32),
            jax.ShapeDtypeStruct((N_PAD, CCH), jnp.float32),
        ],
    )(neighbour_displacements,
      zi.reshape(N_EDGES, 1),
      zj.reshape(N_EDGES, 1),
      zatoms.reshape(N_PAD, 1),
      rad_mix.reshape(N_SPECIES, N_RBF * F),
      table2, Wtd0a, Wtd0b, Wtd1a, Wtd1b)

    zero = jnp.zeros((ROWS_PT, CCH), jnp.float32)
    mesh = plsc.VectorSubcoreMesh(core_axis_name="c", subcore_axis_name="s")
    out = pl.kernel(
        _sc_scatter,
        mesh=mesh,
        compiler_params=pltpu.CompilerParams(use_tc_tiling_on_sc=False),
        out_type=jax.ShapeDtypeStruct((NCH, N_PAD, CCH), jnp.float32),
        scratch_types=[
            pltpu.VMEM((BATCH,), jnp.int32),
            pltpu.VMEM((BATCH, CCH), jnp.float32),
            pltpu.VMEM((TAIL,), jnp.int32),
            pltpu.VMEM((TAIL, CCH), jnp.float32),
            pltpu.VMEM_SHARED((N_PAD, CCH), jnp.float32),
        ],
    )(y, idx_i, res, zero)

    flat = out.transpose(1, 0, 2).reshape(N_PAD, NCH * CCH)
    return flat[:N_ATOMS, :D_OUT].reshape(N_ATOMS, N_SH, F)


# speedup vs baseline: 5.7606x; 1.2081x over previous
import functools

import jax
import jax.numpy as jnp
import numpy as np
from jax import lax
from jax.experimental import pallas as pl
from jax.experimental.pallas import tpu as pltpu
from jax.experimental.pallas import tpu_sc as plsc

N_ATOMS = 10000
N_EDGES = 160000
N_SPECIES = 100
F = 64
N_RBF = 32
N_SH = 9
D_OUT = N_SH * F
CUTOFF = 5.0

BE = 1000
GRID = N_EDGES // BE

NCORES = 2
NSUB = 16
NCH = 5
CCH = 128
N_PAD = 10240
ROWS_PT = N_PAD // NSUB
EPT = N_EDGES // NSUB
BATCH = 128
NSTEP = EPT // BATCH
TAIL = EPT - NSTEP * BATCH


def _sh_channels(x, y, z):
    return [
        jnp.full_like(x, 0.28209479),
        0.48860251 * y,
        0.48860251 * z,
        0.48860251 * x,
        1.09254843 * x * y,
        1.09254843 * y * z,
        0.31539157 * (3.0 * z * z - 1.0),
        1.09254843 * x * z,
        0.54627422 * (x * x - y * y),
    ]


def _edge_kernel(disp_ref, zi_ref, zj_ref, zatoms_ref,
                 mix_ref, table2_ref, w0a_ref, w0b_ref, w1a_ref, w1b_ref,
                 y_ref, res_ref):
    g = pl.program_id(0)

    disp = disp_ref[...]
    dx = disp[:, 0:1]
    dy = disp[:, 1:2]
    dz = disp[:, 2:3]
    r2 = dx * dx + dy * dy + dz * dz
    r = jnp.sqrt(r2)
    inv = 1.0 / (r + 1e-9)
    ux, uy, uz = dx * inv, dy * inv, dz * inv

    env = 0.5 * (jnp.cos(jnp.pi * jnp.clip(r / CUTOFF, 0.0, 1.0)) + 1.0)

    zj = zj_ref[...]
    sp_iota = jax.lax.broadcasted_iota(jnp.int32, (1, N_SPECIES), 1)
    onehot_j = (zj == sp_iota).astype(jnp.float32)
    radial_all = jnp.dot(onehot_j, mix_ref[...],
                         preferred_element_type=jnp.float32)
    lane_b = jax.lax.broadcasted_iota(jnp.int32, (1, N_RBF * F), 1) // F
    centers_exp = lane_b.astype(jnp.float32) * (CUTOFF / (N_RBF - 1))
    d = r - centers_exp
    prod = jnp.exp(-4.0 * d * d) * env * radial_all
    w = N_RBF * F
    while w > F:
        w //= 2
        prod = prod[:, :w] + prod[:, w:2 * w]
    radial = prod

    zi = zi_ref[...]
    onehot_i = (zi == sp_iota).astype(jnp.float32)
    emb_i = jnp.dot(onehot_i, table2_ref[...],
                    preferred_element_type=jnp.float32)

    sh = _sh_channels(ux, uy, uz)
    ys = [sh[c] * radial for c in range(9)]

    for wa_ref, wb_ref in ((w0a_ref, w0b_ref), (w1a_ref, w1b_ref)):
        wa = wa_ref[...]
        wb = wb_ref[...]
        hs = [jnp.dot(yc, wa, preferred_element_type=jnp.float32)
              for yc in ys]
        gate = hs[0] * jax.nn.sigmoid(hs[0])
        ys = [jnp.dot(hc * gate, wb, preferred_element_type=jnp.float32)
              for hc in hs]

    for ch in range(4):
        for j in range(2):
            y_ref[ch, :, j * F:(j + 1) * F] = ys[2 * ch + j] * emb_i
    y_ref[4, :, 0:F] = ys[8] * emb_i
    y_ref[4, :, F:CCH] = jnp.zeros((BE, CCH - F), jnp.float32)

    @pl.when(g == GRID - 1)
    def _residual():
        za = zatoms_ref[...]
        onehot_a = (za == sp_iota).astype(jnp.float32)
        res = jnp.dot(onehot_a, table2_ref[...],
                      preferred_element_type=jnp.float32)
        res_ref[...] = jnp.concatenate(
            [res, jnp.zeros((N_PAD, CCH - F), jnp.float32)], axis=1)


def _sc_scatter(y_hbm, idx_hbm, res_hbm, zero_hbm, out_hbm,
                idx_v, y_v, idx_t, y_t, acc):
    c = lax.axis_index("c")
    s = lax.axis_index("s")
    stripe = pl.ds(s * ROWS_PT, ROWS_PT)

    for k in range(3):
        chunk = c + NCORES * k

        @pl.when(chunk < NCH)
        def _pass():
            @pl.when(chunk == 0)
            def _init_res():
                pltpu.sync_copy(res_hbm.at[stripe], acc.at[stripe])

            @pl.when(chunk != 0)
            def _init_zero():
                pltpu.sync_copy(zero_hbm, acc.at[stripe])

            plsc.subcore_barrier()

            def step(i, _):
                base = s * EPT + i * BATCH
                pltpu.sync_copy(idx_hbm.at[pl.ds(base, BATCH)], idx_v)
                pltpu.sync_copy(y_hbm.at[chunk, pl.ds(base, BATCH)], y_v)
                pltpu.sync_copy(y_v, acc.at[idx_v], add=True)
                return 0
            lax.fori_loop(0, NSTEP, step, 0)

            tbase = s * EPT + NSTEP * BATCH
            pltpu.sync_copy(idx_hbm.at[pl.ds(tbase, TAIL)], idx_t)
            pltpu.sync_copy(y_hbm.at[chunk, pl.ds(tbase, TAIL)], y_t)
            pltpu.sync_copy(y_t, acc.at[idx_t], add=True)

            plsc.subcore_barrier()

            pltpu.sync_copy(acc.at[stripe], out_hbm.at[chunk, stripe])


def kernel(atomic_numbers, neighbour_indices, neighbour_displacements,
           embed_table, W_emb, b_emb, rad_mix, Wtd0a, Wtd0b, Wtd1a, Wtd1b):
    idx_i = neighbour_indices[:, 0].astype(jnp.int32)
    idx_j = neighbour_indices[:, 1]
    zi = jnp.take(atomic_numbers, idx_i, axis=0).astype(jnp.int32)
    zj = jnp.take(atomic_numbers, idx_j, axis=0).astype(jnp.int32)
    table2 = embed_table @ W_emb + b_emb
    zatoms = jnp.concatenate(
        [atomic_numbers.astype(jnp.int32),
         jnp.full((N_PAD - N_ATOMS,), -1, jnp.int32)])

    y, res = pl.pallas_call(
        _edge_kernel,
        grid=(GRID,),
        in_specs=[
            pl.BlockSpec((BE, 3), lambda i: (i, 0)),
            pl.BlockSpec((BE, 1), lambda i: (i, 0)),
            pl.BlockSpec((BE, 1), lambda i: (i, 0)),
            pl.BlockSpec((N_PAD, 1), lambda i: (0, 0)),
            pl.BlockSpec((N_SPECIES, N_RBF * F), lambda i: (0, 0)),
            pl.BlockSpec((N_SPECIES, F), lambda i: (0, 0)),
            pl.BlockSpec((F, F), lambda i: (0, 0)),
            pl.BlockSpec((F, F), lambda i: (0, 0)),
            pl.BlockSpec((F, F), lambda i: (0, 0)),
            pl.BlockSpec((F, F), lambda i: (0, 0)),
        ],
        out_specs=[
            pl.BlockSpec((NCH, BE, CCH), lambda i: (0, i, 0)),
            pl.BlockSpec((N_PAD, CCH), lambda i: (0, 0)),
        ],
        out_shape=[
            jax.ShapeDtypeStruct((NCH, N_EDGES, CCH), jnp.float32),
            jax.ShapeDtypeStruct((N_PAD, CCH), jnp.float32),
        ],
    )(neighbour_displacements,
      zi.reshape(N_EDGES, 1),
      zj.reshape(N_EDGES, 1),
      zatoms.reshape(N_PAD, 1),
      rad_mix.reshape(N_SPECIES, N_RBF * F),
      table2, Wtd0a, Wtd0b, Wtd1a, Wtd1b)

    zero = jnp.zeros((ROWS_PT, CCH), jnp.float32)
    mesh = plsc.VectorSubcoreMesh(core_axis_name="c", subcore_axis_name="s")
    out = pl.kernel(
        _sc_scatter,
        mesh=mesh,
        compiler_params=pltpu.CompilerParams(use_tc_tiling_on_sc=False),
        out_type=jax.ShapeDtypeStruct((NCH, N_PAD, CCH), jnp.float32),
        scratch_types=[
            pltpu.VMEM((BATCH,), jnp.int32),
            pltpu.VMEM((BATCH, CCH), jnp.float32),
            pltpu.VMEM((TAIL,), jnp.int32),
            pltpu.VMEM((TAIL, CCH), jnp.float32),
            pltpu.VMEM_SHARED((N_PAD, CCH), jnp.float32),
        ],
    )(y, idx_i, res, zero)

    flat = out.transpose(1, 0, 2).reshape(N_PAD, NCH * CCH)
    return flat[:N_ATOMS, :D_OUT].reshape(N_ATOMS, N_SH, F)


# speedup vs baseline: 6.0676x; 1.0533x over previous
import functools

import jax
import jax.numpy as jnp
import numpy as np
from jax import lax
from jax.experimental import pallas as pl
from jax.experimental.pallas import tpu as pltpu
from jax.experimental.pallas import tpu_sc as plsc

N_ATOMS = 10000
N_EDGES = 160000
N_SPECIES = 100
F = 64
N_RBF = 32
N_SH = 9
D_OUT = N_SH * F
CUTOFF = 5.0

BE = 1000
GRID = N_EDGES // BE

NCORES = 2
NSUB = 16
NCH = 5
CCH = 128
N_PAD = 10240
ROWS_PT = N_PAD // NSUB
EPT = N_EDGES // NSUB
BATCH = 128
NSTEP = EPT // BATCH
TAIL = EPT - NSTEP * BATCH


def _sh_channels(x, y, z):
    return [
        jnp.full_like(x, 0.28209479),
        0.48860251 * y,
        0.48860251 * z,
        0.48860251 * x,
        1.09254843 * x * y,
        1.09254843 * y * z,
        0.31539157 * (3.0 * z * z - 1.0),
        1.09254843 * x * z,
        0.54627422 * (x * x - y * y),
    ]


def _edge_kernel(disp_ref, zi_ref, zj_ref, zatoms_ref,
                 mix_ref, table2_ref, w0a_ref, w0b_ref, w1a_ref, w1b_ref,
                 y_ref, res_ref):
    g = pl.program_id(0)

    disp = disp_ref[...]
    dx = disp[:, 0:1]
    dy = disp[:, 1:2]
    dz = disp[:, 2:3]
    r2 = dx * dx + dy * dy + dz * dz
    r = jnp.sqrt(r2)
    inv = 1.0 / (r + 1e-9)
    ux, uy, uz = dx * inv, dy * inv, dz * inv

    env = 0.5 * (jnp.cos(jnp.pi * jnp.clip(r / CUTOFF, 0.0, 1.0)) + 1.0)

    zj = zj_ref[...]
    sp_iota = jax.lax.broadcasted_iota(jnp.int32, (1, N_SPECIES), 1)
    onehot_j = (zj == sp_iota).astype(jnp.float32)
    radial_all = jnp.dot(onehot_j, mix_ref[...],
                         preferred_element_type=jnp.float32)
    lane_b = jax.lax.broadcasted_iota(jnp.int32, (1, N_RBF * F), 1) // F
    centers_exp = lane_b.astype(jnp.float32) * (CUTOFF / (N_RBF - 1))
    d = r - centers_exp
    prod = jnp.exp(-4.0 * d * d) * env * radial_all
    w = N_RBF * F
    while w > F:
        w //= 2
        prod = prod[:, :w] + prod[:, w:2 * w]
    radial = prod

    zi = zi_ref[...]
    onehot_i = (zi == sp_iota).astype(jnp.float32)
    emb_i = jnp.dot(onehot_i, table2_ref[...],
                    preferred_element_type=jnp.float32)

    sh = _sh_channels(ux, uy, uz)
    Y = jnp.concatenate([sh[c] * radial for c in range(9)], axis=0)

    for wa_ref, wb_ref in ((w0a_ref, w0b_ref), (w1a_ref, w1b_ref)):
        h = jnp.dot(Y, wa_ref[...], preferred_element_type=jnp.float32)
        g0 = h[0:BE]
        gate = g0 * jax.nn.sigmoid(g0)
        gate9 = jnp.concatenate([gate] * 9, axis=0)
        Y = jnp.dot(h * gate9, wb_ref[...],
                    preferred_element_type=jnp.float32)

    for ch in range(4):
        for j in range(2):
            cc = 2 * ch + j
            y_ref[ch, :, j * F:(j + 1) * F] = (
                Y[cc * BE:(cc + 1) * BE] * emb_i)
    y_ref[4, :, 0:F] = Y[8 * BE:9 * BE] * emb_i
    y_ref[4, :, F:CCH] = jnp.zeros((BE, CCH - F), jnp.float32)

    @pl.when(g == GRID - 1)
    def _residual():
        za = zatoms_ref[...]
        onehot_a = (za == sp_iota).astype(jnp.float32)
        res = jnp.dot(onehot_a, table2_ref[...],
                      preferred_element_type=jnp.float32)
        res_ref[...] = jnp.concatenate(
            [res, jnp.zeros((N_PAD, CCH - F), jnp.float32)], axis=1)


def _sc_scatter(y_hbm, idx_hbm, res_hbm, zero_hbm, out_hbm,
                idx_v, y_v, idx_t, y_t, acc):
    c = lax.axis_index("c")
    s = lax.axis_index("s")
    stripe = pl.ds(s * ROWS_PT, ROWS_PT)

    for k in range(3):
        chunk = c + NCORES * k

        @pl.when(chunk < NCH)
        def _pass():
            @pl.when(chunk == 0)
            def _init_res():
                pltpu.sync_copy(res_hbm.at[stripe], acc.at[stripe])

            @pl.when(chunk != 0)
            def _init_zero():
                pltpu.sync_copy(zero_hbm, acc.at[stripe])

            plsc.subcore_barrier()

            def step(i, _):
                base = s * EPT + i * BATCH
                pltpu.sync_copy(idx_hbm.at[pl.ds(base, BATCH)], idx_v)
                pltpu.sync_copy(y_hbm.at[chunk, pl.ds(base, BATCH)], y_v)
                pltpu.sync_copy(y_v, acc.at[idx_v], add=True)
                return 0
            lax.fori_loop(0, NSTEP, step, 0)

            tbase = s * EPT + NSTEP * BATCH
            pltpu.sync_copy(idx_hbm.at[pl.ds(tbase, TAIL)], idx_t)
            pltpu.sync_copy(y_hbm.at[chunk, pl.ds(tbase, TAIL)], y_t)
            pltpu.sync_copy(y_t, acc.at[idx_t], add=True)

            plsc.subcore_barrier()

            pltpu.sync_copy(acc.at[stripe], out_hbm.at[chunk, stripe])


def kernel(atomic_numbers, neighbour_indices, neighbour_displacements,
           embed_table, W_emb, b_emb, rad_mix, Wtd0a, Wtd0b, Wtd1a, Wtd1b):
    idx_i = neighbour_indices[:, 0].astype(jnp.int32)
    idx_j = neighbour_indices[:, 1]
    zi = jnp.take(atomic_numbers, idx_i, axis=0).astype(jnp.int32)
    zj = jnp.take(atomic_numbers, idx_j, axis=0).astype(jnp.int32)
    table2 = embed_table @ W_emb + b_emb
    zatoms = jnp.concatenate(
        [atomic_numbers.astype(jnp.int32),
         jnp.full((N_PAD - N_ATOMS,), -1, jnp.int32)])

    y, res = pl.pallas_call(
        _edge_kernel,
        grid=(GRID,),
        in_specs=[
            pl.BlockSpec((BE, 3), lambda i: (i, 0)),
            pl.BlockSpec((BE, 1), lambda i: (i, 0)),
            pl.BlockSpec((BE, 1), lambda i: (i, 0)),
            pl.BlockSpec((N_PAD, 1), lambda i: (0, 0)),
            pl.BlockSpec((N_SPECIES, N_RBF * F), lambda i: (0, 0)),
            pl.BlockSpec((N_SPECIES, F), lambda i: (0, 0)),
            pl.BlockSpec((F, F), lambda i: (0, 0)),
            pl.BlockSpec((F, F), lambda i: (0, 0)),
            pl.BlockSpec((F, F), lambda i: (0, 0)),
            pl.BlockSpec((F, F), lambda i: (0, 0)),
        ],
        out_specs=[
            pl.BlockSpec((NCH, BE, CCH), lambda i: (0, i, 0)),
            pl.BlockSpec((N_PAD, CCH), lambda i: (0, 0)),
        ],
        out_shape=[
            jax.ShapeDtypeStruct((NCH, N_EDGES, CCH), jnp.float32),
            jax.ShapeDtypeStruct((N_PAD, CCH), jnp.float32),
        ],
    )(neighbour_displacements,
      zi.reshape(N_EDGES, 1),
      zj.reshape(N_EDGES, 1),
      zatoms.reshape(N_PAD, 1),
      rad_mix.reshape(N_SPECIES, N_RBF * F),
      table2, Wtd0a, Wtd0b, Wtd1a, Wtd1b)

    zero = jnp.zeros((ROWS_PT, CCH), jnp.float32)
    mesh = plsc.VectorSubcoreMesh(core_axis_name="c", subcore_axis_name="s")
    out = pl.kernel(
        _sc_scatter,
        mesh=mesh,
        compiler_params=pltpu.CompilerParams(use_tc_tiling_on_sc=False),
        out_type=jax.ShapeDtypeStruct((NCH, N_PAD, CCH), jnp.float32),
        scratch_types=[
            pltpu.VMEM((BATCH,), jnp.int32),
            pltpu.VMEM((BATCH, CCH), jnp.float32),
            pltpu.VMEM((TAIL,), jnp.int32),
            pltpu.VMEM((TAIL, CCH), jnp.float32),
            pltpu.VMEM_SHARED((N_PAD, CCH), jnp.float32),
        ],
    )(y, idx_i, res, zero)

    flat = out.transpose(1, 0, 2).reshape(N_PAD, NCH * CCH)
    return flat[:N_ATOMS, :D_OUT].reshape(N_ATOMS, N_SH, F)
